# Initial kernel scaffold; baseline (speedup 1.0000x reference)
import jax
import jax.numpy as jnp
from jax.experimental import pallas as pl


def kernel(input_ids, word_embeddings):
    raise NotImplementedError("write your pallas kernel here")


# speedup vs baseline: 1.7730x; 1.7730x over previous
import functools

import jax
import jax.numpy as jnp
from jax import lax
from jax.experimental import pallas as pl
from jax.experimental.pallas import tpu as pltpu
from jax.experimental.pallas import tpu_sc as plsc

NUM_CORES = 2
NUM_SUBCORES = 16
NUM_WORKERS = NUM_CORES * NUM_SUBCORES

CHUNK = 32


@functools.partial(jax.jit, static_argnames=())
def _gather_rows(flat_idx, table):
    n = flat_idx.shape[0]
    d = table.shape[1]
    n_per_w = n // NUM_WORKERS
    n_chunks = n_per_w // CHUNK

    mesh = plsc.VectorSubcoreMesh(core_axis_name="c", subcore_axis_name="s")

    @functools.partial(
        pl.kernel,
        mesh=mesh,
        out_type=jax.ShapeDtypeStruct((n, d), jnp.float32),
        scratch_types=[
            pltpu.VMEM((n_per_w,), jnp.int32),
            pltpu.VMEM((CHUNK, d), jnp.float32),
            pltpu.VMEM((CHUNK, d), jnp.float32),
            pltpu.SemaphoreType.DMA,
            pltpu.SemaphoreType.DMA,
        ],
    )
    def k(idx_hbm, table_hbm, out_hbm, idx_v, rows0, rows1, sem0, sem1):
        wid = lax.axis_index("s") * NUM_CORES + lax.axis_index("c")
        base = wid * n_per_w
        pltpu.sync_copy(idx_hbm.at[pl.ds(base, n_per_w)], idx_v)

        bufs = (rows0, rows1)
        sems = (sem0, sem1)

        pltpu.async_copy(table_hbm.at[idx_v.at[pl.ds(0, CHUNK)]], rows0, sem0)

        def body(i, carry):
            @pl.when(i + 1 < n_chunks)
            def _start():
                for b in range(2):

                    @pl.when(lax.rem(i + 1, 2) == b)
                    def _():
                        pltpu.async_copy(
                            table_hbm.at[idx_v.at[pl.ds((i + 1) * CHUNK, CHUNK)]],
                            bufs[b],
                            sems[b],
                        )

            for b in range(2):

                @pl.when(lax.rem(i, 2) == b)
                def _():
                    pltpu.make_async_copy(
                        table_hbm.at[idx_v.at[pl.ds(i * CHUNK, CHUNK)]],
                        bufs[b],
                        sems[b],
                    ).wait()
                    pltpu.sync_copy(bufs[b], out_hbm.at[pl.ds(base + i * CHUNK, CHUNK)])

            return carry

        lax.fori_loop(0, n_chunks, body, 0)

    return k(flat_idx, table)


def kernel(input_ids, word_embeddings):
    b, s = input_ids.shape
    d = word_embeddings.shape[1]
    flat_idx = input_ids.reshape(b * s).astype(jnp.int32)
    out = _gather_rows(flat_idx, word_embeddings)
    return out.reshape(b, s, d)


# speedup vs baseline: 1.7890x; 1.0090x over previous
import functools

import jax
import jax.numpy as jnp
from jax import lax
from jax.experimental import pallas as pl
from jax.experimental.pallas import tpu as pltpu
from jax.experimental.pallas import tpu_sc as plsc

NUM_CORES = 2
NUM_SUBCORES = 16
NUM_WORKERS = NUM_CORES * NUM_SUBCORES

CHUNK = 32


@functools.partial(jax.jit, static_argnames=())
def _gather_rows(flat_idx, table):
    n = flat_idx.shape[0]
    d = table.shape[1]
    n_per_w = n // NUM_WORKERS
    n_chunks = n_per_w // CHUNK

    mesh = plsc.VectorSubcoreMesh(core_axis_name="c", subcore_axis_name="s")
    NBUF = 3

    @functools.partial(
        pl.kernel,
        mesh=mesh,
        out_type=jax.ShapeDtypeStruct((n, d), jnp.float32),
        scratch_types=[
            pltpu.VMEM((n_per_w,), jnp.int32),
            *[pltpu.VMEM((CHUNK, d), jnp.float32) for _ in range(NBUF)],
            *[pltpu.SemaphoreType.DMA for _ in range(2 * NBUF)],
        ],
    )
    def k(idx_hbm, table_hbm, out_hbm, idx_v, *bufs_and_sems):
        bufs = bufs_and_sems[:NBUF]
        gsems = bufs_and_sems[NBUF : 2 * NBUF]
        wsems = bufs_and_sems[2 * NBUF :]

        wid = lax.axis_index("s") * NUM_CORES + lax.axis_index("c")
        base = wid * n_per_w
        pltpu.sync_copy(idx_hbm.at[pl.ds(base, n_per_w)], idx_v)

        def gather(i, b):
            pltpu.async_copy(
                table_hbm.at[idx_v.at[pl.ds(i * CHUNK, CHUNK)]], bufs[b], gsems[b]
            )

        def wait_gather(i, b):
            pltpu.make_async_copy(
                table_hbm.at[idx_v.at[pl.ds(i * CHUNK, CHUNK)]], bufs[b], gsems[b]
            ).wait()

        def write(i, b):
            pltpu.async_copy(
                bufs[b], out_hbm.at[pl.ds(base + i * CHUNK, CHUNK)], wsems[b]
            )

        def wait_write(i, b):
            pltpu.make_async_copy(
                bufs[b], out_hbm.at[pl.ds(base + i * CHUNK, CHUNK)], wsems[b]
            ).wait()

        for j in range(NBUF - 1):
            gather(j, j)

        def body(i, carry):
            @pl.when(i >= 1)
            def _():
                for b in range(NBUF):

                    @pl.when(lax.rem(i - 1, NBUF) == b)
                    def _():
                        wait_write(i - 1, b)

            @pl.when(i + NBUF - 1 < n_chunks)
            def _():
                for b in range(NBUF):

                    @pl.when(lax.rem(i + NBUF - 1, NBUF) == b)
                    def _():
                        gather(i + NBUF - 1, b)

            for b in range(NBUF):

                @pl.when(lax.rem(i, NBUF) == b)
                def _():
                    wait_gather(i, b)
                    write(i, b)

            return carry

        lax.fori_loop(0, n_chunks, body, 0)
        wait_write(n_chunks - 1, (n_chunks - 1) % NBUF)

    return k(flat_idx, table)


def kernel(input_ids, word_embeddings):
    b, s = input_ids.shape
    d = word_embeddings.shape[1]
    flat_idx = input_ids.reshape(b * s).astype(jnp.int32)
    out = _gather_rows(flat_idx, word_embeddings)
    return out.reshape(b, s, d)
